# Initial kernel scaffold; baseline (speedup 1.0000x reference)
import jax
import jax.numpy as jnp
from jax.experimental import pallas as pl


def kernel(xus, xut, xvs, xvt, edge_index, xe_e, xe_v2u, xe_u2v, Wu, bu, gu, beu, Wv, bv, gv, bev, We, be, ge, bee):
    raise NotImplementedError("write your pallas kernel here")


# speedup vs baseline: 1.0230x; 1.0230x over previous
import functools

import jax
import jax.numpy as jnp
from jax import lax
from jax.experimental import pallas as pl
from jax.experimental.pallas import tpu as pltpu

N_U = 10000
N_V = 10000
N_E = 320000
D_N = 128
D_E = 16
EPS = 1e-5


def _node_assemble(x_ref, s_ref, m_ref, es_ref, em_ref, cnt_ref,
                   w1, w2, w3, w4, w5, b_ref, g_ref, be_ref, out_ref):
    cnt = cnt_ref[...][:, :1]
    inv = 1.0 / jnp.maximum(cnt, 1.0)
    nonempty = cnt > 0.0
    mn = s_ref[...] * inv
    mx = jnp.where(nonempty, m_ref[...], 0.0)
    emn = es_ref[...] * inv
    emx = jnp.where(nonempty, em_ref[...], 0.0)
    y = (jnp.dot(x_ref[...], w1[...], preferred_element_type=jnp.float32)
         + jnp.dot(mn, w2[...], preferred_element_type=jnp.float32)
         + jnp.dot(mx, w3[...], preferred_element_type=jnp.float32)
         + jnp.dot(emn, w4[...], preferred_element_type=jnp.float32)
         + jnp.dot(emx, w5[...], preferred_element_type=jnp.float32)
         + b_ref[...])
    mu = jnp.mean(y, axis=0, keepdims=True)
    var = jnp.mean(jnp.square(y - mu), axis=0, keepdims=True)
    out_ref[...] = g_ref[...] * (y - mu) * lax.rsqrt(var + EPS) + be_ref[...]


def _node_out(x, s, m, es, em, cnt, W, b, g, be):
    w1 = W[:, 0:128].T
    w2 = W[:, 128:256].T
    w3 = W[:, 256:384].T
    w4 = W[:, 384:400].T
    w5 = W[:, 400:416].T
    n = x.shape[0]
    return pl.pallas_call(
        _node_assemble,
        out_shape=jax.ShapeDtypeStruct((n, 128), jnp.float32),
    )(x, s, m, es, em, cnt, w1, w2, w3, w4, w5,
      b.reshape(1, 128), g.reshape(1, 128), be.reshape(1, 128))


_EBLK = 4000


def _edge_pass1(xe_ref, pu_ref, pv_ref, bd_ref, bias_ref, ye_ref, stats_ref):
    i = pl.program_id(0)
    y = (jnp.dot(xe_ref[...], bd_ref[...], preferred_element_type=jnp.float32)
         + pu_ref[...] + pv_ref[...] + bias_ref[...])
    ye_ref[...] = y

    @pl.when(i == 0)
    def _():
        stats_ref[...] = jnp.zeros_like(stats_ref)

    s = jnp.sum(y, axis=0, keepdims=True)
    ss = jnp.sum(y * y, axis=0, keepdims=True)
    stats_ref[0:1, :] += s
    stats_ref[1:2, :] += ss


def _edge_pass2(ye_ref, scale_ref, shift_ref, out_ref):
    out_ref[...] = ye_ref[...] * scale_ref[...] + shift_ref[...]


def _edge_out(xe_f, pu_f, pv_f, We_e, be, ge, bee):
    bd = jnp.kron(jnp.eye(8, dtype=jnp.float32), We_e.T)
    bias = jnp.tile(be, 8).reshape(1, 128)
    nrows = N_E // 8
    nblk = nrows // _EBLK
    ye, stats = pl.pallas_call(
        _edge_pass1,
        grid=(nblk,),
        in_specs=[
            pl.BlockSpec((_EBLK, 128), lambda i: (i, 0)),
            pl.BlockSpec((_EBLK, 128), lambda i: (i, 0)),
            pl.BlockSpec((_EBLK, 128), lambda i: (i, 0)),
            pl.BlockSpec((128, 128), lambda i: (0, 0)),
            pl.BlockSpec((1, 128), lambda i: (0, 0)),
        ],
        out_specs=[
            pl.BlockSpec((_EBLK, 128), lambda i: (i, 0)),
            pl.BlockSpec((8, 128), lambda i: (0, 0)),
        ],
        out_shape=[
            jax.ShapeDtypeStruct((nrows, 128), jnp.float32),
            jax.ShapeDtypeStruct((8, 128), jnp.float32),
        ],
    )(xe_f, pu_f, pv_f, bd, bias)
    s16 = jnp.sum(stats[0].reshape(8, 16), axis=0)
    ss16 = jnp.sum(stats[1].reshape(8, 16), axis=0)
    mu = s16 / N_E
    var = ss16 / N_E - mu * mu
    scale = ge * lax.rsqrt(var + EPS)
    shift = bee - mu * scale
    out = pl.pallas_call(
        _edge_pass2,
        grid=(nblk,),
        in_specs=[
            pl.BlockSpec((_EBLK, 128), lambda i: (i, 0)),
            pl.BlockSpec((1, 128), lambda i: (0, 0)),
            pl.BlockSpec((1, 128), lambda i: (0, 0)),
        ],
        out_specs=pl.BlockSpec((_EBLK, 128), lambda i: (i, 0)),
        out_shape=jax.ShapeDtypeStruct((nrows, 128), jnp.float32),
    )(ye, jnp.tile(scale, 8).reshape(1, 128), jnp.tile(shift, 8).reshape(1, 128))
    return out.reshape(N_E, 16)


def _segments(vals, evals, idx, n):
    s = jax.ops.segment_sum(vals, idx, num_segments=n)
    cnt = jax.ops.segment_sum(jnp.ones((vals.shape[0],), jnp.float32), idx,
                              num_segments=n)
    m = jax.ops.segment_max(vals, idx, num_segments=n)
    es = jax.ops.segment_sum(evals, idx, num_segments=n)
    em = jax.ops.segment_max(evals, idx, num_segments=n)
    cnt16 = jnp.broadcast_to(cnt[:, None], (n, 16))
    return s, m, es, em, cnt16


def kernel(xus, xut, xvs, xvt, edge_index, xe_e, xe_v2u, xe_u2v,
           Wu, bu, gu, beu, Wv, bv, gv, bev, We, be, ge, bee):
    row = jnp.asarray(edge_index[0], jnp.int32)
    col = jnp.asarray(edge_index[1], jnp.int32)

    s_u, m_u, es_u, em_u, cnt_u = _segments(xvs[col], xe_v2u, row, N_U)
    s_v, m_v, es_v, em_v, cnt_v = _segments(xus[row], xe_u2v, col, N_V)

    out_u = _node_out(xut, s_u, m_u, es_u, em_u, cnt_u, Wu, bu, gu, beu)
    out_v = _node_out(xvt, s_v, m_v, es_v, em_v, cnt_v, Wv, bv, gv, bev)

    pu = jnp.dot(xut, We[:, 16:144].T)
    pv = jnp.dot(xvt, We[:, 144:272].T)
    pu_e = pu[row]
    pv_e = pv[col]
    out_e = _edge_out(xe_e.reshape(N_E // 8, 128),
                      pu_e.reshape(N_E // 8, 128),
                      pv_e.reshape(N_E // 8, 128),
                      We[:, 0:16], be, ge, bee)
    return (out_u, out_v, out_e)


# speedup vs baseline: 1.7890x; 1.7488x over previous
import functools

import jax
import jax.numpy as jnp
from jax import lax
from jax.experimental import pallas as pl
from jax.experimental.pallas import tpu as pltpu
from jax.experimental.pallas import tpu_sc as plsc

N_U = 10000
N_V = 10000
N_E = 320000
D_N = 128
D_E = 16
EPS = 1e-5

NC = 2
NS = 16
NW = NC * NS
RPT = 320
NPAD = NW * RPT
TRASH = RPT
ACC_R = RPT + 8
CHUNK = 1280
NCHUNKS = N_E // CHUNK
GRPS = CHUNK // 16
GB = 96
CB = 256
NEG_INF = float("-inf")


def _seg_body(row_hbm, col_hbm, xvs_hbm, xus_hbm, xev_hbm, xeu_hbm,
              su_hbm, mu_hbm, auxu_hbm, sv_hbm, mv_hbm, auxv_hbm,
              rowstage, colstage, ccol, cdst, ceid, gcol, geid,
              vals, evals, acc_s, acc_m, acc_es, acc_em, acc_c,
              off_ref, sem1, sem2):
    cid = lax.axis_index("c")
    sid = lax.axis_index("s")
    wid = sid * NC + cid
    lo = wid * RPT

    zeros16 = jnp.zeros((16,), jnp.float32)
    minf16 = jnp.full((16,), NEG_INF, jnp.float32)

    def accumulate_batch():
        def acc_one(j, carry):
            d = cdst[pl.ds(j, 16)][0]
            e = ceid[pl.ds(j, 16)][0]
            for cg in range(8):
                sl = pl.ds(cg * 16, 16)
                v = vals[j, sl]
                acc_s[d, sl] = acc_s[d, sl] + v
                acc_m[d, sl] = jnp.maximum(acc_m[d, sl], v)
            ev = evals[j, pl.ds((e & 7) * 16, 16)]
            d16 = pl.ds(d * 16, 16)
            acc_es[d16] = acc_es[d16] + ev
            acc_em[d16] = jnp.maximum(acc_em[d16], ev)
            acc_c[d16] = acc_c[d16] + 1.0
            return carry
        lax.fori_loop(0, GB, acc_one, 0)

    def run_direction(dst_hbm, src_hbm, table_hbm, etable_hbm,
                      s_hbm, m_hbm, aux_hbm):
        def init_one(r, carry):
            for cg in range(8):
                sl = pl.ds(cg * 16, 16)
                acc_s[r, sl] = zeros16
                acc_m[r, sl] = minf16
            r16 = pl.ds(r * 16, 16)
            acc_es[r16] = zeros16
            acc_em[r16] = minf16
            acc_c[r16] = zeros16
            return carry
        lax.fori_loop(0, ACC_R, init_one, 0)
        off_ref[0] = 0

        def fire():
            for p in range(GB // 16):
                sl = pl.ds(p * 16, 16)
                gcol[sl] = ccol[sl]
                geid[sl] = ceid[sl] >> 3
            d1 = pltpu.async_copy(table_hbm.at[gcol], vals, sem1)
            d2 = pltpu.async_copy(etable_hbm.at[geid], evals, sem2)
            d1.wait()
            d2.wait()
            accumulate_batch()

        def chunk_body(k, carry):
            base = k * CHUNK
            pltpu.sync_copy(dst_hbm.at[pl.ds(base, CHUNK)], rowstage)
            pltpu.sync_copy(src_hbm.at[pl.ds(base, CHUNK)], colstage)

            def grp_body(g, carry2):
                goff = g * 16
                r = rowstage[pl.ds(goff, 16)]
                c = colstage[pl.ds(goff, 16)]
                m = (r >= lo) & (r < lo + RPT)
                off = off_ref[0]
                pfx = jnp.cumsum(jnp.where(m, jnp.full((16,), 1, jnp.int32), jnp.zeros((16,), jnp.int32)))
                idx = jnp.maximum(off + pfx - 1, 0)
                plsc.store_scatter(ccol, [idx], c, mask=m)
                plsc.store_scatter(cdst, [idx], r - lo, mask=m)
                eidv = lax.iota(jnp.int32, 16) + (base + goff)
                plsc.store_scatter(ceid, [idx], eidv, mask=m)
                off_ref[0] = off + pfx[15]

                @pl.when(off_ref[0] >= GB)
                def _():
                    fire()
                    rem_c = ccol[pl.ds(GB, 16)]
                    rem_d = cdst[pl.ds(GB, 16)]
                    rem_e = ceid[pl.ds(GB, 16)]
                    ccol[pl.ds(0, 16)] = rem_c
                    cdst[pl.ds(0, 16)] = rem_d
                    ceid[pl.ds(0, 16)] = rem_e
                    off_ref[0] = off_ref[0] - GB
                return carry2

            lax.fori_loop(0, GRPS, grp_body, 0)
            return carry

        lax.fori_loop(0, NCHUNKS, chunk_body, 0)

        off = off_ref[0]
        for p in range(GB // 16):
            sl = pl.ds(p * 16, 16)
            lanes = lax.iota(jnp.int32, 16) + (p * 16)
            valid = lanes < off
            ccol[sl] = jnp.where(valid, ccol[sl], 0)
            ceid[sl] = jnp.where(valid, ceid[sl], 0)
            cdst[sl] = jnp.where(valid, cdst[sl], TRASH)
        fire()

        pltpu.sync_copy(acc_s.at[pl.ds(0, RPT)], s_hbm.at[pl.ds(lo, RPT)])
        pltpu.sync_copy(acc_m.at[pl.ds(0, RPT)], m_hbm.at[pl.ds(lo, RPT)])
        pltpu.sync_copy(acc_es.at[pl.ds(0, RPT * 16)],
                        aux_hbm.at[pl.ds(lo * 16, RPT * 16)])
        pltpu.sync_copy(acc_em.at[pl.ds(0, RPT * 16)],
                        aux_hbm.at[pl.ds(NPAD * 16 + lo * 16, RPT * 16)])
        pltpu.sync_copy(acc_c.at[pl.ds(0, RPT * 16)],
                        aux_hbm.at[pl.ds(2 * NPAD * 16 + lo * 16, RPT * 16)])

    run_direction(row_hbm, col_hbm, xvs_hbm, xev_hbm,
                  su_hbm, mu_hbm, auxu_hbm)
    run_direction(col_hbm, row_hbm, xus_hbm, xeu_hbm,
                  sv_hbm, mv_hbm, auxv_hbm)


def _sc_segments(row, col, xvs, xus, xe_v2u_f, xe_u2v_f):
    mesh = plsc.VectorSubcoreMesh(core_axis_name="c", subcore_axis_name="s")
    nf = jnp.float32
    out_type = [
        jax.ShapeDtypeStruct((NPAD, 128), nf),
        jax.ShapeDtypeStruct((NPAD, 128), nf),
        jax.ShapeDtypeStruct((3 * NPAD * 16,), nf),
        jax.ShapeDtypeStruct((NPAD, 128), nf),
        jax.ShapeDtypeStruct((NPAD, 128), nf),
        jax.ShapeDtypeStruct((3 * NPAD * 16,), nf),
    ]
    scratch = [
        pltpu.VMEM((CHUNK,), jnp.int32),
        pltpu.VMEM((CHUNK,), jnp.int32),
        pltpu.VMEM((CB,), jnp.int32),
        pltpu.VMEM((CB,), jnp.int32),
        pltpu.VMEM((CB,), jnp.int32),
        pltpu.VMEM((GB,), jnp.int32),
        pltpu.VMEM((GB,), jnp.int32),
        pltpu.VMEM((GB, 128), nf),
        pltpu.VMEM((GB, 128), nf),
        pltpu.VMEM((ACC_R, 128), nf),
        pltpu.VMEM((ACC_R, 128), nf),
        pltpu.VMEM((ACC_R * 16,), nf),
        pltpu.VMEM((ACC_R * 16,), nf),
        pltpu.VMEM((ACC_R * 16,), nf),
        pltpu.SMEM((8,), jnp.int32),
        pltpu.SemaphoreType.DMA,
        pltpu.SemaphoreType.DMA,
    ]
    run = pl.kernel(_seg_body, out_type=out_type, mesh=mesh,
                    scratch_types=scratch,
                    compiler_params=pltpu.CompilerParams(
                        needs_layout_passes=False))
    return run(row, col, xvs, xus, xe_v2u_f, xe_u2v_f)


_EPT = N_E // NW
_GBATCH = 2000
_NGB = _EPT // _GBATCH


def _puv_body(row_hbm, col_hbm, pu_hbm, pv_hbm, outu_hbm, outv_hbm,
              ridx, cidx, bufu, bufv, sem1, sem2):
    cid = lax.axis_index("c")
    sid = lax.axis_index("s")
    wid = sid * NC + cid

    def batch_body(b, carry):
        base = wid * _EPT + b * _GBATCH
        pltpu.sync_copy(row_hbm.at[pl.ds(base, _GBATCH)], ridx)
        pltpu.sync_copy(col_hbm.at[pl.ds(base, _GBATCH)], cidx)
        d1 = pltpu.async_copy(pu_hbm.at[ridx], bufu, sem1)
        d2 = pltpu.async_copy(pv_hbm.at[cidx], bufv, sem2)
        d1.wait()
        d2.wait()
        pltpu.sync_copy(bufu, outu_hbm.at[pl.ds(base, _GBATCH)])
        pltpu.sync_copy(bufv, outv_hbm.at[pl.ds(base, _GBATCH)])
        return carry

    lax.fori_loop(0, _NGB, batch_body, 0)


def _sc_puv(row, col, pu, pv):
    mesh = plsc.VectorSubcoreMesh(core_axis_name="c", subcore_axis_name="s")
    run = pl.kernel(
        _puv_body,
        out_type=[jax.ShapeDtypeStruct((N_E, 16), jnp.float32),
                  jax.ShapeDtypeStruct((N_E, 16), jnp.float32)],
        mesh=mesh,
        scratch_types=[
            pltpu.VMEM((_GBATCH,), jnp.int32),
            pltpu.VMEM((_GBATCH,), jnp.int32),
            pltpu.VMEM((_GBATCH, 16), jnp.float32),
            pltpu.VMEM((_GBATCH, 16), jnp.float32),
            pltpu.SemaphoreType.DMA,
            pltpu.SemaphoreType.DMA,
        ],
        compiler_params=pltpu.CompilerParams(use_tc_tiling_on_sc=False),
    )
    return run(row, col, pu, pv)


def _node_assemble(x_ref, s_ref, m_ref, es_ref, em_ref, cnt_ref,
                   w1, w2, w3, w4, w5, b_ref, g_ref, be_ref, out_ref):
    cnt = cnt_ref[...][:, :1]
    inv = 1.0 / jnp.maximum(cnt, 1.0)
    nonempty = cnt > 0.0
    mn = s_ref[...] * inv
    mx = jnp.where(nonempty, m_ref[...], 0.0)
    emn = es_ref[...] * inv
    emx = jnp.where(nonempty, em_ref[...], 0.0)
    y = (jnp.dot(x_ref[...], w1[...], preferred_element_type=jnp.float32)
         + jnp.dot(mn, w2[...], preferred_element_type=jnp.float32)
         + jnp.dot(mx, w3[...], preferred_element_type=jnp.float32)
         + jnp.dot(emn, w4[...], preferred_element_type=jnp.float32)
         + jnp.dot(emx, w5[...], preferred_element_type=jnp.float32)
         + b_ref[...])
    mu = jnp.mean(y, axis=0, keepdims=True)
    var = jnp.mean(jnp.square(y - mu), axis=0, keepdims=True)
    out_ref[...] = g_ref[...] * (y - mu) * lax.rsqrt(var + EPS) + be_ref[...]


def _node_out(x, s, m, es, em, cnt, W, b, g, be):
    w1 = W[:, 0:128].T
    w2 = W[:, 128:256].T
    w3 = W[:, 256:384].T
    w4 = W[:, 384:400].T
    w5 = W[:, 400:416].T
    n = x.shape[0]
    return pl.pallas_call(
        _node_assemble,
        out_shape=jax.ShapeDtypeStruct((n, 128), jnp.float32),
    )(x, s, m, es, em, cnt, w1, w2, w3, w4, w5,
      b.reshape(1, 128), g.reshape(1, 128), be.reshape(1, 128))


def _proj_kernel(xu_ref, xv_ref, wu_ref, wv_ref, pu_ref, pv_ref):
    pu_ref[...] = jnp.dot(xu_ref[...], wu_ref[...],
                          preferred_element_type=jnp.float32)
    pv_ref[...] = jnp.dot(xv_ref[...], wv_ref[...],
                          preferred_element_type=jnp.float32)


def _projections(xut, xvt, We):
    return pl.pallas_call(
        _proj_kernel,
        out_shape=[jax.ShapeDtypeStruct((N_U, 16), jnp.float32),
                   jax.ShapeDtypeStruct((N_V, 16), jnp.float32)],
    )(xut, xvt, We[:, 16:144].T, We[:, 144:272].T)


_EBLK = 4000


def _edge_pass1(xe_ref, pu_ref, pv_ref, bd_ref, bias_ref, ye_ref, stats_ref):
    i = pl.program_id(0)
    y = (jnp.dot(xe_ref[...], bd_ref[...], preferred_element_type=jnp.float32)
         + pu_ref[...] + pv_ref[...] + bias_ref[...])
    ye_ref[...] = y

    @pl.when(i == 0)
    def _():
        stats_ref[...] = jnp.zeros_like(stats_ref)

    s = jnp.sum(y, axis=0, keepdims=True)
    ss = jnp.sum(y * y, axis=0, keepdims=True)
    stats_ref[0:1, :] += s
    stats_ref[1:2, :] += ss


def _edge_pass2(ye_ref, scale_ref, shift_ref, out_ref):
    out_ref[...] = ye_ref[...] * scale_ref[...] + shift_ref[...]


def _edge_out(xe_f, pu_f, pv_f, We_e, be, ge, bee):
    bd = jnp.kron(jnp.eye(8, dtype=jnp.float32), We_e.T)
    bias = jnp.tile(be, 8).reshape(1, 128)
    nrows = N_E // 8
    nblk = nrows // _EBLK
    ye, stats = pl.pallas_call(
        _edge_pass1,
        grid=(nblk,),
        in_specs=[
            pl.BlockSpec((_EBLK, 128), lambda i: (i, 0)),
            pl.BlockSpec((_EBLK, 128), lambda i: (i, 0)),
            pl.BlockSpec((_EBLK, 128), lambda i: (i, 0)),
            pl.BlockSpec((128, 128), lambda i: (0, 0)),
            pl.BlockSpec((1, 128), lambda i: (0, 0)),
        ],
        out_specs=[
            pl.BlockSpec((_EBLK, 128), lambda i: (i, 0)),
            pl.BlockSpec((8, 128), lambda i: (0, 0)),
        ],
        out_shape=[
            jax.ShapeDtypeStruct((nrows, 128), jnp.float32),
            jax.ShapeDtypeStruct((8, 128), jnp.float32),
        ],
    )(xe_f, pu_f, pv_f, bd, bias)
    s16 = jnp.sum(stats[0].reshape(8, 16), axis=0)
    ss16 = jnp.sum(stats[1].reshape(8, 16), axis=0)
    mu = s16 / N_E
    var = ss16 / N_E - mu * mu
    scale = ge * lax.rsqrt(var + EPS)
    shift = bee - mu * scale
    out = pl.pallas_call(
        _edge_pass2,
        grid=(nblk,),
        in_specs=[
            pl.BlockSpec((_EBLK, 128), lambda i: (i, 0)),
            pl.BlockSpec((1, 128), lambda i: (0, 0)),
            pl.BlockSpec((1, 128), lambda i: (0, 0)),
        ],
        out_specs=pl.BlockSpec((_EBLK, 128), lambda i: (i, 0)),
        out_shape=jax.ShapeDtypeStruct((nrows, 128), jnp.float32),
    )(ye, jnp.tile(scale, 8).reshape(1, 128), jnp.tile(shift, 8).reshape(1, 128))
    return out.reshape(N_E, 16)


def kernel(xus, xut, xvs, xvt, edge_index, xe_e, xe_v2u, xe_u2v,
           Wu, bu, gu, beu, Wv, bv, gv, bev, We, be, ge, bee):
    row = jnp.asarray(edge_index[0], jnp.int32)
    col = jnp.asarray(edge_index[1], jnp.int32)

    (s_u, m_u, aux_u,
     s_v, m_v, aux_v) = _sc_segments(row, col, xvs, xus,
                                     xe_v2u.reshape(N_E // 8, 128),
                                     xe_u2v.reshape(N_E // 8, 128))

    def unaux(aux):
        es = aux[0:NPAD * 16].reshape(NPAD, 16)
        em = aux[NPAD * 16:2 * NPAD * 16].reshape(NPAD, 16)
        c = aux[2 * NPAD * 16:].reshape(NPAD, 16)
        return es[:N_U], em[:N_U], c[:N_U]

    es_u, em_u, cnt_u = unaux(aux_u)
    es_v, em_v, cnt_v = unaux(aux_v)

    out_u = _node_out(xut, s_u[:N_U], m_u[:N_U], es_u, em_u, cnt_u,
                      Wu, bu, gu, beu)
    out_v = _node_out(xvt, s_v[:N_V], m_v[:N_V], es_v, em_v, cnt_v,
                      Wv, bv, gv, bev)

    pu, pv = _projections(xut, xvt, We)
    pu_e, pv_e = _sc_puv(row, col, pu, pv)
    out_e = _edge_out(xe_e.reshape(N_E // 8, 128),
                      pu_e.reshape(N_E // 8, 128),
                      pv_e.reshape(N_E // 8, 128),
                      We[:, 0:16], be, ge, bee)
    return (out_u, out_v, out_e)
